# baseline (speedup 1.0000x reference)
import jax
import jax.numpy as jnp
from jax.experimental import pallas as pl
from jax.experimental.pallas import tpu as pltpu

N = 16384
D = 64
BR = 1024
G = N // BR


def _tc_body(x_hbm, psum_ref, xv, sem):
    g = pl.program_id(0)
    cp = pltpu.make_async_copy(x_hbm.at[pl.ds(g * BR, BR)], xv, sem)
    cp.start()
    cp.wait()
    psum_ref[0, ...] = jnp.sum(xv[...].reshape(BR // 8, 8, D), axis=0)


def _tc_kernel(x):
    return pl.pallas_call(
        _tc_body,
        grid=(G,),
        in_specs=[pl.BlockSpec(memory_space=pl.ANY)],
        out_specs=[pl.BlockSpec((1, 8, D), lambda g: (g, 0, 0))],
        out_shape=[jax.ShapeDtypeStruct((G, 8, D), jnp.float32)],
        scratch_shapes=[
            pltpu.VMEM((BR, D), jnp.float32),
            pltpu.SemaphoreType.DMA,
        ],
        compiler_params=pltpu.CompilerParams(
            dimension_semantics=("arbitrary",),
        ),
    )(x)


def kernel(x):
    (psums,) = _tc_kernel(x)
    return psums.sum()


# speedup vs baseline: 2.3663x; 2.3663x over previous
import jax
import jax.numpy as jnp
from jax.experimental import pallas as pl
from jax.experimental.pallas import tpu as pltpu

N = 16384
D = 64
BC = 2048
G = N // BC


def _tc_body(x_ref, add_ref, mul_ref, psum_ref):
    g = pl.program_id(0)
    x = x_ref[...]
    rowc = jax.lax.broadcasted_iota(jnp.int32, (D, BC), 1).astype(jnp.float32) + (
        (g * BC).astype(jnp.float32) + 2.0)
    add_ref[...] = x + rowc
    mul_ref[...] = x * 3.0
    psum_ref[0, 0, :] = jnp.sum(x.reshape(D * BC // 128, 128), axis=0)


def _tc_kernel(xt):
    return pl.pallas_call(
        _tc_body,
        grid=(G,),
        in_specs=[pl.BlockSpec((D, BC), lambda g: (0, g))],
        out_specs=[
            pl.BlockSpec((D, BC), lambda g: (0, g)),
            pl.BlockSpec((D, BC), lambda g: (0, g)),
            pl.BlockSpec((1, 1, 128), lambda g: (g, 0, 0)),
        ],
        out_shape=[
            jax.ShapeDtypeStruct((D, N), jnp.float32),
            jax.ShapeDtypeStruct((D, N), jnp.float32),
            jax.ShapeDtypeStruct((G, 1, 128), jnp.float32),
        ],
        compiler_params=pltpu.CompilerParams(
            dimension_semantics=("arbitrary",),
        ),
    )(xt)


def kernel(x):
    add_t, mul_t, psums = _tc_kernel(x.T)
    mean_result = psums.sum() / (N * D) + (2.0 + (N - 1) / 2.0)
    return (add_t.T, mul_t.T, mean_result)


# speedup vs baseline: 3.0895x; 1.3056x over previous
import jax
import jax.numpy as jnp
from jax.experimental import pallas as pl
from jax.experimental.pallas import tpu as pltpu

N = 16384
D = 64
BC = 2048
G = N // BC


def _tc_body(x_ref, add_ref, mul_ref, mean_ref, rowc_ref, acc_ref):
    g = pl.program_id(0)

    @pl.when(g == 0)
    def _init():
        rowc_ref[...] = jax.lax.broadcasted_iota(
            jnp.int32, (D, BC), 1).astype(jnp.float32) + 2.0
        acc_ref[...] = jnp.zeros((8, 128), jnp.float32)

    x = x_ref[...]
    add_ref[...] = (x + rowc_ref[...]) + (g * BC).astype(jnp.float32)
    mul_ref[...] = x * 3.0
    acc_ref[...] += jnp.sum(x.reshape(D * BC // (8 * 128), 8, 128), axis=0)

    @pl.when(g == G - 1)
    def _fin():
        total = jnp.sum(acc_ref[...])
        mean_ref[0, 0] = total / (N * D) + (2.0 + (N - 1) / 2.0)


def _tc_kernel(xt):
    return pl.pallas_call(
        _tc_body,
        grid=(G,),
        in_specs=[pl.BlockSpec((D, BC), lambda g: (0, g))],
        out_specs=[
            pl.BlockSpec((D, BC), lambda g: (0, g)),
            pl.BlockSpec((D, BC), lambda g: (0, g)),
            pl.BlockSpec(memory_space=pltpu.SMEM),
        ],
        out_shape=[
            jax.ShapeDtypeStruct((D, N), jnp.float32),
            jax.ShapeDtypeStruct((D, N), jnp.float32),
            jax.ShapeDtypeStruct((1, 1), jnp.float32),
        ],
        scratch_shapes=[
            pltpu.VMEM((D, BC), jnp.float32),
            pltpu.VMEM((8, 128), jnp.float32),
        ],
        compiler_params=pltpu.CompilerParams(
            dimension_semantics=("arbitrary",),
        ),
    )(xt)


def kernel(x):
    add_t, mul_t, mean2d = _tc_kernel(x.T)
    return (add_t.T, mul_t.T, mean2d.reshape(()))


# speedup vs baseline: 3.8876x; 1.2583x over previous
import jax
import jax.numpy as jnp
from jax.experimental import pallas as pl
from jax.experimental.pallas import tpu as pltpu

N = 16384
D = 64
BC = 4096
G = N // BC


def _tc_body(x_ref, add_ref, mul_ref, mean_ref, rowc_ref, acc_ref):
    g = pl.program_id(0)

    @pl.when(g == 0)
    def _init():
        rowc_ref[...] = jax.lax.broadcasted_iota(
            jnp.int32, (D, BC), 1).astype(jnp.float32) + 2.0
        acc_ref[...] = jnp.zeros((8, 128), jnp.float32)

    x = x_ref[...]
    add_ref[...] = (x + rowc_ref[...]) + (g * BC).astype(jnp.float32)
    mul_ref[...] = x * 3.0
    acc_ref[...] += jnp.sum(x.reshape(D * BC // (8 * 128), 8, 128), axis=0)

    @pl.when(g == G - 1)
    def _fin():
        total = jnp.sum(acc_ref[...])
        mean_ref[0, 0] = total / (N * D) + (2.0 + (N - 1) / 2.0)


def _tc_kernel(xt):
    return pl.pallas_call(
        _tc_body,
        grid=(G,),
        in_specs=[pl.BlockSpec((D, BC), lambda g: (0, g))],
        out_specs=[
            pl.BlockSpec((D, BC), lambda g: (0, g)),
            pl.BlockSpec((D, BC), lambda g: (0, g)),
            pl.BlockSpec(memory_space=pltpu.SMEM),
        ],
        out_shape=[
            jax.ShapeDtypeStruct((D, N), jnp.float32),
            jax.ShapeDtypeStruct((D, N), jnp.float32),
            jax.ShapeDtypeStruct((1, 1), jnp.float32),
        ],
        scratch_shapes=[
            pltpu.VMEM((D, BC), jnp.float32),
            pltpu.VMEM((8, 128), jnp.float32),
        ],
        compiler_params=pltpu.CompilerParams(
            dimension_semantics=("arbitrary",),
        ),
    )(xt)


def kernel(x):
    add_t, mul_t, mean2d = _tc_kernel(x.T)
    return (add_t.T, mul_t.T, mean2d.reshape(()))


# speedup vs baseline: 4.4318x; 1.1400x over previous
import jax
import jax.numpy as jnp
from jax.experimental import pallas as pl
from jax.experimental.pallas import tpu as pltpu

N = 16384
D = 64
BC = 8192
G = N // BC


def _tc_body(x_ref, add_ref, mul_ref, mean_ref, rowc_ref, acc_ref):
    g = pl.program_id(0)

    @pl.when(g == 0)
    def _init():
        rowc_ref[...] = jax.lax.broadcasted_iota(
            jnp.int32, (D, BC), 1).astype(jnp.float32) + 2.0
        acc_ref[...] = jnp.zeros((8, 128), jnp.float32)

    x = x_ref[...]
    add_ref[...] = (x + rowc_ref[...]) + (g * BC).astype(jnp.float32)
    mul_ref[...] = x * 3.0
    acc_ref[...] += jnp.sum(x.reshape(D * BC // (8 * 128), 8, 128), axis=0)

    @pl.when(g == G - 1)
    def _fin():
        total = jnp.sum(acc_ref[...])
        mean_ref[0, 0] = total / (N * D) + (2.0 + (N - 1) / 2.0)


def _tc_kernel(xt):
    return pl.pallas_call(
        _tc_body,
        grid=(G,),
        in_specs=[pl.BlockSpec((D, BC), lambda g: (0, g))],
        out_specs=[
            pl.BlockSpec((D, BC), lambda g: (0, g)),
            pl.BlockSpec((D, BC), lambda g: (0, g)),
            pl.BlockSpec(memory_space=pltpu.SMEM),
        ],
        out_shape=[
            jax.ShapeDtypeStruct((D, N), jnp.float32),
            jax.ShapeDtypeStruct((D, N), jnp.float32),
            jax.ShapeDtypeStruct((1, 1), jnp.float32),
        ],
        scratch_shapes=[
            pltpu.VMEM((D, BC), jnp.float32),
            pltpu.VMEM((8, 128), jnp.float32),
        ],
        compiler_params=pltpu.CompilerParams(
            dimension_semantics=("arbitrary",),
        ),
    )(xt)


def kernel(x):
    add_t, mul_t, mean2d = _tc_kernel(x.T)
    return (add_t.T, mul_t.T, mean2d.reshape(()))
